# Initial kernel scaffold; baseline (speedup 1.0000x reference)
import jax
import jax.numpy as jnp
from jax.experimental import pallas as pl


def kernel(y_pred_0, y_pred_1, y_pred_2, y_true):
    raise NotImplementedError("write your pallas kernel here")


# speedup vs baseline: 1.7801x; 1.7801x over previous
import math

import jax
import jax.numpy as jnp
from jax import lax
from jax.experimental import pallas as pl
from jax.experimental.pallas import tpu as pltpu
from jax.experimental.pallas import tpu_sc as plsc

_B, _L, _NCLS = 16, 50, 80
_STRIDES = (8.0, 16.0, 32.0)
_HW = ((64, 64), (32, 32), (16, 16))
_ANCHORS = ((12.0, 16.0), (19.0, 36.0), (40.0, 28.0),
            (36.0, 75.0), (76.0, 55.0), (72.0, 146.0),
            (142.0, 110.0), (192.0, 243.0), (459.0, 401.0))
_ANC = tuple(tuple((_ANCHORS[li * 3 + a][0] / _STRIDES[li],
                    _ANCHORS[li * 3 + a][1] / _STRIDES[li]) for a in range(3))
             for li in range(3))
_MATCH = 4.0
_WBOX, _WOBJ, _WCLS = 0.05, 1.0, 0.58

_NT = _B * 3
_CAP = 256
_FCAP = 288
_ROWS = _NT * _CAP
_CW = 96
_XR = 16
_FP0, _FP1, _FP2, _FP3, _FP4 = 0, 1, 2, 3, 4
_FCELL, _FGCX, _FGCY, _FGW, _FGH, _FLAB, _FWIN, _FVAL = 5, 6, 7, 8, 9, 10, 11, 12


def _sc_task(t, lane, yps, yt_hbm, gouts, xouts, ytv, baseA, cellA, ordA,
             gcxA, gcyA, gwA, ghA, labA, mapA, idxA, gathA, gathT, auxA,
             cellS, valS, jgcx, jgcy, jgw, jgh, jlab, sem_g, sem_w):
  b = t // 3
  a = t % 3
  truem = lane >= jnp.minimum(t, 0)

  pltpu.sync_copy(yt_hbm.at[b], ytv)

  xs, ys, ws, hs, labs, valids, gcxs, gcys = [], [], [], [], [], [], [], []
  for v in range(4):
    j5 = (lane + v * 16) * 5
    x = plsc.load_gather(ytv, [j5], mask=truem)
    y = plsc.load_gather(ytv, [j5 + 1], mask=truem)
    w = plsc.load_gather(ytv, [j5 + 2], mask=truem)
    h = plsc.load_gather(ytv, [j5 + 3], mask=truem)
    lb = plsc.load_gather(ytv, [j5 + 4], mask=truem)
    xs.append(x); ys.append(y); ws.append(w); hs.append(h); labs.append(lb)
    valids.append((x + y + w + h + lb) > 0.0)
    gcxs.append(x + w * 0.5)
    gcys.append(y + h * 0.5)
    jslot = lane + v * 16
    plsc.store_scatter(jgcx, [jslot], gcxs[v], mask=truem)
    plsc.store_scatter(jgcy, [jslot], gcys[v], mask=truem)
    plsc.store_scatter(jgw, [jslot], ws[v], mask=truem)
    plsc.store_scatter(jgh, [jslot], hs[v], mask=truem)
    plsc.store_scatter(jlab, [jslot], labs[v], mask=truem)

  for li in range(3):
    s = _STRIDES[li]
    H, W = _HW[li]
    HWp = H * W
    yp = yps[li]
    aw = jnp.where(a == 0, _ANC[li][0][0],
                   jnp.where(a == 1, _ANC[li][1][0], _ANC[li][2][0]))
    ah = jnp.where(a == 0, _ANC[li][0][1],
                   jnp.where(a == 1, _ANC[li][1][1], _ANC[li][2][1]))
    base0 = b * (255 * HWp) + a * (85 * HWp)

    ms, sxs, sys_ = [], [], []
    for v in range(4):
      r1 = ws[v] / (s * aw)
      r2 = hs[v] / (s * ah)
      maxr = jnp.maximum(jnp.maximum(r1, 1.0 / r1), jnp.maximum(r2, 1.0 / r2))
      ms.append((maxr < _MATCH) & valids[v])
      sxs.append(gcxs[v] / s)
      sys_.append(gcys[v] / s)

    for o in range(5):
      for v in range(4):
        sx, sy, m = sxs[v], sys_[v], ms[v]
        fx = sx - sx.astype(jnp.int32).astype(jnp.float32)
        fy = sy - sy.astype(jnp.int32).astype(jnp.float32)
        if o == 0:
          cond = m; cx = sx; cy = sy
        elif o == 1:
          cond = m & (sx > 1.0) & (fx < 0.5); cx = sx - 0.5; cy = sy
        elif o == 2:
          cond = m & (sy > 1.0) & (fy < 0.5); cx = sx; cy = sy - 0.5
        elif o == 3:
          cond = m & (sx < W - 1.0) & (fx > 0.5); cx = sx + 0.5; cy = sy
        else:
          cond = m & (sy < H - 1.0) & (fy > 0.5); cx = sx; cy = sy + 0.5
        gxi = jnp.clip(cx.astype(jnp.int32), 0, W - 1)
        gyi = jnp.clip(cy.astype(jnp.int32), 0, H - 1)
        cell = gyi * W + gxi
        slot = lane + (o * 4 + v) * 16
        plsc.store_scatter(cellS, [slot], cell, mask=cond)
        plsc.store_scatter(valS, [slot], cond.astype(jnp.int32), mask=truem)

    def compact_body(g, cnt_c):
      vldv = valS[pl.ds(g * 16, 16)]
      valid = vldv > 0
      cellv = cellS[pl.ds(g * 16, 16)]
      ordv = g * 16 + lane
      jv = lax.rem(ordv, 64)
      rel = plsc.cumsum(valid.astype(jnp.int32))
      pos = cnt_c + rel - 1
      plsc.store_scatter(baseA, [pos], base0 + cellv, mask=valid)
      plsc.store_scatter(cellA, [pos], cellv, mask=valid)
      plsc.store_scatter(ordA, [pos], ordv, mask=valid)
      plsc.store_scatter(gcxA, [pos], plsc.load_gather(jgcx, [jv], mask=valid),
                         mask=valid)
      plsc.store_scatter(gcyA, [pos], plsc.load_gather(jgcy, [jv], mask=valid),
                         mask=valid)
      plsc.store_scatter(gwA, [pos], plsc.load_gather(jgw, [jv], mask=valid),
                         mask=valid)
      plsc.store_scatter(ghA, [pos], plsc.load_gather(jgh, [jv], mask=valid),
                         mask=valid)
      plsc.store_scatter(labA, [pos], plsc.load_gather(jlab, [jv], mask=valid),
                         mask=valid)
      return cnt_c + jnp.max(rel)
    n = lax.fori_loop(0, 20, compact_body, jnp.int32(0))
    nb = (n + 15) // 16

    def win_body(k, c):
      iv = lane + k * 16
      msk = iv < n
      cellv = cellA[pl.ds(k * 16, 16)]
      ov = ordA[pl.ds(k * 16, 16)]
      plsc.store_scatter(mapA, [cellv], ov, mask=msk)
      return c
    lax.fori_loop(0, nb, win_body, 0)

    def build_body(k, c):
      iv = lane + k * 16
      msk = iv < n
      cellv = cellA[pl.ds(k * 16, 16)]
      ov = ordA[pl.ds(k * 16, 16)]
      bv = baseA[pl.ds(k * 16, 16)]
      winv = (plsc.load_gather(mapA, [cellv], mask=msk) == ov) & msk
      plsc.store_scatter(auxA, [5 * _CAP + iv], cellv.astype(jnp.float32),
                         mask=truem)
      plsc.store_scatter(auxA, [6 * _CAP + iv], winv.astype(jnp.float32),
                         mask=truem)
      bv_safe = jnp.where(msk, bv, 0)
      for e in range(_CW):
        plsc.store_scatter(idxA, [iv * _CW + e], bv_safe + min(e, 84) * HWp,
                           mask=truem)
      return c
    lax.fori_loop(0, nb, build_body, 0)

    for q in range(_CAP // 16):
      iv = lane + q * 16
      plsc.store_scatter(auxA, [7 * _CAP + iv], (iv < n).astype(jnp.float32),
                         mask=truem)

    def fire_body(k, c):
      for cc in range(16):
        i = k * 16 + cc
        pltpu.async_copy(yp.at[idxA.at[pl.ds(i * _CW, _CW)]],
                         gathA.at[pl.ds(i * _CW, _CW)], sem_g)
      return c
    lax.fori_loop(0, nb, fire_body, 0)

    def drain_body(k, c):
      for cc in range(16):
        pltpu.make_async_copy(yp.at[idxA.at[pl.ds(0, _CW)]],
                              gathA.at[pl.ds(0, _CW)], sem_g).wait()
      return c
    lax.fori_loop(0, nb, drain_body, 0)

    def tr_body(k, c):
      iv = lane + k * 16
      for f in range(_CW):
        pv = plsc.load_gather(gathA, [iv * _CW + f], mask=truem)
        plsc.store_scatter(gathT, [f * _CAP + iv], pv, mask=truem)
        if f < 5:
          plsc.store_scatter(auxA, [f * _CAP + iv], pv, mask=truem)
      return c
    lax.fori_loop(0, nb, tr_body, 0)

    gout, xout = gouts[li], xouts[li]
    pltpu.async_copy(gathT, gout.at[pl.ds(t * _CW * _CAP, _CW * _CAP)], sem_w)

    col = t * _CAP
    aux_srcs = [auxA.at[pl.ds(0 * _CAP, _CAP)],
                auxA.at[pl.ds(1 * _CAP, _CAP)],
                auxA.at[pl.ds(2 * _CAP, _CAP)],
                auxA.at[pl.ds(3 * _CAP, _CAP)],
                auxA.at[pl.ds(4 * _CAP, _CAP)],
                auxA.at[pl.ds(5 * _CAP, _CAP)],
                gcxA.at[pl.ds(0, _CAP)],
                gcyA.at[pl.ds(0, _CAP)],
                gwA.at[pl.ds(0, _CAP)],
                ghA.at[pl.ds(0, _CAP)],
                labA.at[pl.ds(0, _CAP)],
                auxA.at[pl.ds(6 * _CAP, _CAP)],
                auxA.at[pl.ds(7 * _CAP, _CAP)]]
    for fr, src in enumerate(aux_srcs):
      pltpu.async_copy(src, xout.at[pl.ds(fr * _ROWS + col, _CAP)], sem_w)
    for src in aux_srcs:
      pltpu.make_async_copy(src, xout.at[pl.ds(0, _CAP)], sem_w).wait()
    pltpu.make_async_copy(gathT, gout.at[pl.ds(0, _CW * _CAP)], sem_w).wait()


def _sc_kernel_body(yp0, yp1, yp2, yt_hbm, lanes_hbm, g0, g1, g2, x0, x1, x2,
                    ytv, baseA, cellA, ordA, gcxA, gcyA, gwA, ghA, labA,
                    mapA, idxA, gathA, gathT, auxA, laneA,
                    cellS, valS, jgcx, jgcy, jgw, jgh, jlab, sem_g, sem_w):
  wid = lax.axis_index("s") * 2 + lax.axis_index("c")
  pltpu.sync_copy(lanes_hbm, laneA.at[pl.ds(0, 16)])
  lane = laneA[pl.ds(0, 16)]
  yps = (yp0, yp1, yp2)
  gouts = (g0, g1, g2)
  xouts = (x0, x1, x2)
  args = (lane, yps, yt_hbm, gouts, xouts, ytv, baseA, cellA, ordA,
          gcxA, gcyA, gwA, ghA, labA, mapA, idxA, gathA, gathT, auxA,
          cellS, valS, jgcx, jgcy, jgw, jgh, jlab, sem_g, sem_w)
  _sc_task(wid, *args)
  @pl.when(wid < _NT - 32)
  def _():
    _sc_task(wid + 32, *args)


def _sc_call(yp0f, yp1f, yp2f, yt_pad, lanes):
  mesh = plsc.VectorSubcoreMesh(core_axis_name="c", subcore_axis_name="s")
  out_type = [jax.ShapeDtypeStruct((_ROWS * _CW,), jnp.float32)
              for _ in range(3)]
  out_type += [jax.ShapeDtypeStruct((_XR * _ROWS,), jnp.float32)
               for _ in range(3)]
  f = pl.kernel(
      _sc_kernel_body,
      out_type=out_type,
      mesh=mesh,
      compiler_params=pltpu.CompilerParams(needs_layout_passes=False),
      scratch_types=[
          pltpu.VMEM((320,), jnp.float32),
          pltpu.VMEM((_FCAP,), jnp.int32),
          pltpu.VMEM((_FCAP,), jnp.int32),
          pltpu.VMEM((_FCAP,), jnp.int32),
          pltpu.VMEM((_FCAP,), jnp.float32),
          pltpu.VMEM((_FCAP,), jnp.float32),
          pltpu.VMEM((_FCAP,), jnp.float32),
          pltpu.VMEM((_FCAP,), jnp.float32),
          pltpu.VMEM((_FCAP,), jnp.float32),
          pltpu.VMEM((4096,), jnp.int32),
          pltpu.VMEM((_CAP * _CW,), jnp.int32),
          pltpu.VMEM((_CAP * _CW,), jnp.float32),
          pltpu.VMEM((_CW * _CAP,), jnp.float32),
          pltpu.VMEM((8 * _CAP,), jnp.float32),
          pltpu.VMEM((128,), jnp.int32),
          pltpu.VMEM((320,), jnp.int32),
          pltpu.VMEM((320,), jnp.int32),
          pltpu.VMEM((64,), jnp.float32),
          pltpu.VMEM((64,), jnp.float32),
          pltpu.VMEM((64,), jnp.float32),
          pltpu.VMEM((64,), jnp.float32),
          pltpu.VMEM((64,), jnp.float32),
          pltpu.SemaphoreType.DMA,
          pltpu.SemaphoreType.DMA,
      ],
  )
  return f(yp0f, yp1f, yp2f, yt_pad, lanes)


def _atan_pos(z):
  inv = z > 1.0
  t = jnp.where(inv, 1.0 / z, z)
  t2 = t * t
  p = 0.0218612288 + t2 * -0.0040540580
  p = -0.0559098861 + t2 * p
  p = 0.0964200441 + t2 * p
  p = -0.1390853351 + t2 * p
  p = 0.1994653599 + t2 * p
  p = -0.3332985605 + t2 * p
  p = 0.9999993329 + t2 * p
  p = t * p
  return jnp.where(inv, (math.pi / 2) - p, p)


def _softplus(x):
  return jnp.maximum(x, 0.0) + jnp.log1p(jnp.exp(-jnp.abs(x)))


def _tc_body(yp0, yp1, yp2, g0, g1, g2, x0, x1, x2, out):
  t = pl.program_id(0)
  a = t % 3

  @pl.when(t == 0)
  def _():
    out[...] = jnp.zeros_like(out)

  acc = jnp.zeros((8, 128), jnp.float32)
  ri = lax.broadcasted_iota(jnp.int32, (8, 128), 0)
  ci = lax.broadcasted_iota(jnp.int32, (8, 128), 1)
  grefs = (g0, g1, g2)
  xrefs = (x0, x1, x2)
  yrefs = (yp0, yp1, yp2)
  eps = 1e-9
  for li in range(3):
    s = _STRIDES[li]
    H, W = _HW[li]
    A = xrefs[li][...]
    G = grefs[li][0]
    def row(f):
      return A[f:f + 1, :]
    v = row(_FVAL) > 0.5
    vf = v.astype(jnp.float32)
    p0 = jnp.where(v, row(_FP0), 0.0)
    p1 = jnp.where(v, row(_FP1), 0.0)
    p2 = jnp.where(v, row(_FP2), 0.0)
    p3 = jnp.where(v, row(_FP3), 0.0)
    p4 = jnp.where(v, row(_FP4), 0.0)
    cell = jnp.where(v, row(_FCELL), 0.0).astype(jnp.int32)
    gcx = jnp.where(v, row(_FGCX), 0.0) / s
    gcy = jnp.where(v, row(_FGCY), 0.0) / s
    gw = jnp.where(v, row(_FGW), 1.0) / s
    gh = jnp.where(v, row(_FGH), 1.0) / s
    lab = jnp.where(v, row(_FLAB), 0.0)
    win = jnp.where(v, row(_FWIN), 0.0)
    gxf = (cell & (W - 1)).astype(jnp.float32)
    gyf = (cell >> int(math.log2(W))).astype(jnp.float32)
    aw = jnp.where(a == 0, _ANC[li][0][0],
                   jnp.where(a == 1, _ANC[li][1][0], _ANC[li][2][0]))
    ah = jnp.where(a == 0, _ANC[li][0][1],
                   jnp.where(a == 1, _ANC[li][1][1], _ANC[li][2][1]))
    px = 1.0 / (1.0 + jnp.exp(-p0)) + gxf
    py = 1.0 / (1.0 + jnp.exp(-p1)) + gyf
    pw = jnp.exp(p2) * aw
    ph = jnp.exp(p3) * ah
    b1x1 = px - pw * 0.5; b1x2 = px + pw * 0.5
    b1y1 = py - ph * 0.5; b1y2 = py + ph * 0.5
    b2x1 = gcx - gw * 0.5; b2x2 = gcx + gw * 0.5
    b2y1 = gcy - gh * 0.5; b2y2 = gcy + gh * 0.5
    inter = (jnp.maximum(jnp.minimum(b1x2, b2x2) - jnp.maximum(b1x1, b2x1), 0.0)
             * jnp.maximum(jnp.minimum(b1y2, b2y2) - jnp.maximum(b1y1, b2y1),
                           0.0))
    w1 = b1x2 - b1x1; h1 = b1y2 - b1y1 + eps
    w2 = b2x2 - b2x1; h2 = b2y2 - b2y1 + eps
    union = w1 * h1 + w2 * h2 - inter + eps
    iou = inter / union
    cw_ = jnp.maximum(b1x2, b2x2) - jnp.minimum(b1x1, b2x1)
    ch_ = jnp.maximum(b1y2, b2y2) - jnp.minimum(b1y1, b2y1)
    c2 = cw_ * cw_ + ch_ * ch_ + eps
    rho2 = ((b2x1 + b2x2 - b1x1 - b1x2) ** 2
            + (b2y1 + b2y2 - b1y1 - b1y2) ** 2) / 4.0
    vv = (4.0 / (math.pi ** 2)) * (_atan_pos(w2 / h2) - _atan_pos(w1 / h1)) ** 2
    alpha = vv / (1.0 + eps - iou + vv)
    ciou = iou - (rho2 / c2 + vv * alpha)

    box_sum = jnp.sum(vf * (1.0 - ciou))
    cnt = jnp.sum(vf)
    val = jnp.maximum(ciou, 0.0)
    corr = jnp.sum(vf * win * val * p4)

    chan = lax.broadcasted_iota(jnp.int32, (_CW, _CAP), 0)
    cmask = (chan >= 5) & (chan < 85) & v
    gx_ = jnp.where(cmask, G, 0.0)
    sp = jnp.where(cmask, _softplus(gx_), 0.0)
    pick = jnp.where(cmask & ((chan - 5) == lab.astype(jnp.int32)), gx_, 0.0)
    cls_sum = jnp.sum(sp) - jnp.sum(pick)

    plane = yrefs[li][0, 0]
    spo = jnp.sum(_softplus(plane))

    for k, sv in enumerate((box_sum, cnt, cls_sum, corr, spo)):
      acc = acc + jnp.where((ri == li) & (ci == k), sv, 0.0)
  out[...] += acc


def _tc_call(yp0, yp1, yp2, gs, xs):
  grid = (_NT,)
  in_specs = [
      pl.BlockSpec((1, 1, 64, 64), lambda t: (t // 3, 4 + 85 * (t % 3), 0, 0)),
      pl.BlockSpec((1, 1, 32, 32), lambda t: (t // 3, 4 + 85 * (t % 3), 0, 0)),
      pl.BlockSpec((1, 1, 16, 16), lambda t: (t // 3, 4 + 85 * (t % 3), 0, 0)),
      pl.BlockSpec((1, _CW, _CAP), lambda t: (t, 0, 0)),
      pl.BlockSpec((1, _CW, _CAP), lambda t: (t, 0, 0)),
      pl.BlockSpec((1, _CW, _CAP), lambda t: (t, 0, 0)),
      pl.BlockSpec((_XR, _CAP), lambda t: (0, t)),
      pl.BlockSpec((_XR, _CAP), lambda t: (0, t)),
      pl.BlockSpec((_XR, _CAP), lambda t: (0, t)),
  ]
  out_spec = pl.BlockSpec((8, 128), lambda t: (0, 0))
  return pl.pallas_call(
      _tc_body,
      grid=grid,
      in_specs=in_specs,
      out_specs=out_spec,
      out_shape=jax.ShapeDtypeStruct((8, 128), jnp.float32),
      compiler_params=pltpu.CompilerParams(
          dimension_semantics=("arbitrary",)),
  )(yp0, yp1, yp2, gs[0].reshape(_NT, _CW, _CAP), gs[1].reshape(_NT, _CW, _CAP),
    gs[2].reshape(_NT, _CW, _CAP), xs[0].reshape(_XR, _ROWS),
    xs[1].reshape(_XR, _ROWS), xs[2].reshape(_XR, _ROWS))


def kernel(y_pred_0, y_pred_1, y_pred_2, y_true):
  yt = y_true.reshape(_B, _L * 5)
  yt_pad = jnp.pad(yt, ((0, 0), (0, 320 - _L * 5)))
  lanes = jnp.arange(16, dtype=jnp.int32)
  outs = _sc_call(y_pred_0.reshape(-1), y_pred_1.reshape(-1),
                  y_pred_2.reshape(-1), yt_pad, lanes)
  gs, xs = outs[:3], outs[3:]
  acc = _tc_call(y_pred_0, y_pred_1, y_pred_2, gs, xs)

  loss_box = jnp.float32(0.0)
  loss_obj = jnp.float32(0.0)
  loss_cls = jnp.float32(0.0)
  for li in range(3):
    H, W = _HW[li]
    box_sum = acc[li, 0]
    cnt = acc[li, 1]
    cls_sum = acc[li, 2]
    corr = acc[li, 3]
    spo = acc[li, 4]
    cntf = jnp.maximum(cnt, 1.0)
    has = cnt > 0
    loss_box = loss_box + jnp.where(has, box_sum / cntf, 0.0)
    loss_cls = loss_cls + jnp.where(has, cls_sum / (cntf * _NCLS), 0.0)
    loss_obj = loss_obj + (spo - corr) / (_B * 3 * H * W)
  total = _WBOX * loss_box + _WOBJ * loss_obj + _WCLS * loss_cls
  return total, _WBOX * loss_box, _WOBJ * loss_obj, _WCLS * loss_cls


# speedup vs baseline: 2.2340x; 1.2550x over previous
import math

import jax
import jax.numpy as jnp
from jax import lax
from jax.experimental import pallas as pl
from jax.experimental.pallas import tpu as pltpu
from jax.experimental.pallas import tpu_sc as plsc

_B, _L, _NCLS = 16, 50, 80
_STRIDES = (8.0, 16.0, 32.0)
_HW = ((64, 64), (32, 32), (16, 16))
_ANCHORS = ((12.0, 16.0), (19.0, 36.0), (40.0, 28.0),
            (36.0, 75.0), (76.0, 55.0), (72.0, 146.0),
            (142.0, 110.0), (192.0, 243.0), (459.0, 401.0))
_ANC = tuple(tuple((_ANCHORS[li * 3 + a][0] / _STRIDES[li],
                    _ANCHORS[li * 3 + a][1] / _STRIDES[li]) for a in range(3))
             for li in range(3))
_MATCH = 4.0
_WBOX, _WOBJ, _WCLS = 0.05, 1.0, 0.58

_NT = _B * 3
_CAP = 256
_FCAP = 288
_ROWS = _NT * _CAP
_CW = 96
_XR = 16
_FP0, _FP1, _FP2, _FP3, _FP4 = 0, 1, 2, 3, 4
_FCELL, _FGCX, _FGCY, _FGW, _FGH, _FLAB, _FWIN, _FVAL = 5, 6, 7, 8, 9, 10, 11, 12


def _sc_task(t, lane, yps, yt_hbm, gouts, xouts, ytv, baseA, cellA, ordA,
             gcxA, gcyA, gwA, ghA, labA, mapA, idxA, gathA, gathT, auxA,
             cellS, valS, jgcx, jgcy, jgw, jgh, jlab, sem_g, sem_w):
  b = t // 3
  a = t % 3
  truem = lane >= jnp.minimum(t, 0)

  pltpu.sync_copy(yt_hbm.at[b], ytv)

  xs, ys, ws, hs, labs, valids, gcxs, gcys = [], [], [], [], [], [], [], []
  for v in range(4):
    j5 = (lane + v * 16) * 5
    x = plsc.load_gather(ytv, [j5], mask=truem)
    y = plsc.load_gather(ytv, [j5 + 1], mask=truem)
    w = plsc.load_gather(ytv, [j5 + 2], mask=truem)
    h = plsc.load_gather(ytv, [j5 + 3], mask=truem)
    lb = plsc.load_gather(ytv, [j5 + 4], mask=truem)
    xs.append(x); ys.append(y); ws.append(w); hs.append(h); labs.append(lb)
    valids.append((x + y + w + h + lb) > 0.0)
    gcxs.append(x + w * 0.5)
    gcys.append(y + h * 0.5)
    jslot = lane + v * 16
    plsc.store_scatter(jgcx, [jslot], gcxs[v], mask=truem)
    plsc.store_scatter(jgcy, [jslot], gcys[v], mask=truem)
    plsc.store_scatter(jgw, [jslot], ws[v], mask=truem)
    plsc.store_scatter(jgh, [jslot], hs[v], mask=truem)
    plsc.store_scatter(jlab, [jslot], labs[v], mask=truem)

  for li in range(3):
    s = _STRIDES[li]
    H, W = _HW[li]
    HWp = H * W
    yp = yps[li]
    aw = jnp.where(a == 0, _ANC[li][0][0],
                   jnp.where(a == 1, _ANC[li][1][0], _ANC[li][2][0]))
    ah = jnp.where(a == 0, _ANC[li][0][1],
                   jnp.where(a == 1, _ANC[li][1][1], _ANC[li][2][1]))
    base0 = b * (255 * HWp) + a * (85 * HWp)

    ms, sxs, sys_ = [], [], []
    for v in range(4):
      r1 = ws[v] / (s * aw)
      r2 = hs[v] / (s * ah)
      maxr = jnp.maximum(jnp.maximum(r1, 1.0 / r1), jnp.maximum(r2, 1.0 / r2))
      ms.append((maxr < _MATCH) & valids[v])
      sxs.append(gcxs[v] / s)
      sys_.append(gcys[v] / s)

    for o in range(5):
      for v in range(4):
        sx, sy, m = sxs[v], sys_[v], ms[v]
        fx = sx - sx.astype(jnp.int32).astype(jnp.float32)
        fy = sy - sy.astype(jnp.int32).astype(jnp.float32)
        if o == 0:
          cond = m; cx = sx; cy = sy
        elif o == 1:
          cond = m & (sx > 1.0) & (fx < 0.5); cx = sx - 0.5; cy = sy
        elif o == 2:
          cond = m & (sy > 1.0) & (fy < 0.5); cx = sx; cy = sy - 0.5
        elif o == 3:
          cond = m & (sx < W - 1.0) & (fx > 0.5); cx = sx + 0.5; cy = sy
        else:
          cond = m & (sy < H - 1.0) & (fy > 0.5); cx = sx; cy = sy + 0.5
        gxi = jnp.clip(cx.astype(jnp.int32), 0, W - 1)
        gyi = jnp.clip(cy.astype(jnp.int32), 0, H - 1)
        cell = gyi * W + gxi
        slot = lane + (o * 4 + v) * 16
        plsc.store_scatter(cellS, [slot], cell, mask=cond)
        plsc.store_scatter(valS, [slot], cond.astype(jnp.int32), mask=truem)

    def compact_body(g, cnt_c):
      vldv = valS[pl.ds(g * 16, 16)]
      valid = vldv > 0
      cellv = cellS[pl.ds(g * 16, 16)]
      ordv = g * 16 + lane
      jv = lax.rem(ordv, 64)
      rel = plsc.cumsum(valid.astype(jnp.int32))
      pos = cnt_c + rel - 1
      plsc.store_scatter(baseA, [pos], base0 + cellv, mask=valid)
      plsc.store_scatter(cellA, [pos], cellv, mask=valid)
      plsc.store_scatter(ordA, [pos], ordv, mask=valid)
      plsc.store_scatter(gcxA, [pos], plsc.load_gather(jgcx, [jv], mask=valid),
                         mask=valid)
      plsc.store_scatter(gcyA, [pos], plsc.load_gather(jgcy, [jv], mask=valid),
                         mask=valid)
      plsc.store_scatter(gwA, [pos], plsc.load_gather(jgw, [jv], mask=valid),
                         mask=valid)
      plsc.store_scatter(ghA, [pos], plsc.load_gather(jgh, [jv], mask=valid),
                         mask=valid)
      plsc.store_scatter(labA, [pos], plsc.load_gather(jlab, [jv], mask=valid),
                         mask=valid)
      return cnt_c + jnp.max(rel)
    n = lax.fori_loop(0, 20, compact_body, jnp.int32(0))
    nb = (n + 15) // 16

    def win_body(k, c):
      iv = lane + k * 16
      msk = iv < n
      cellv = cellA[pl.ds(k * 16, 16)]
      ov = ordA[pl.ds(k * 16, 16)]
      plsc.store_scatter(mapA, [cellv], ov, mask=msk)
      return c
    lax.fori_loop(0, nb, win_body, 0)

    def build_body(k, c):
      iv = lane + k * 16
      msk = iv < n
      cellv = cellA[pl.ds(k * 16, 16)]
      ov = ordA[pl.ds(k * 16, 16)]
      bv = baseA[pl.ds(k * 16, 16)]
      winv = (plsc.load_gather(mapA, [cellv], mask=msk) == ov) & msk
      plsc.store_scatter(auxA, [5 * _CAP + iv], cellv.astype(jnp.float32),
                         mask=truem)
      plsc.store_scatter(auxA, [6 * _CAP + iv], winv.astype(jnp.float32),
                         mask=truem)
      bv_safe = jnp.where(msk, bv, 0)
      for e in range(_CW):
        plsc.store_scatter(idxA, [iv * _CW + e], bv_safe + min(e, 84) * HWp,
                           mask=truem)
      return c
    lax.fori_loop(0, nb, build_body, 0)

    for q in range(_CAP // 16):
      iv = lane + q * 16
      plsc.store_scatter(auxA, [7 * _CAP + iv], (iv < n).astype(jnp.float32),
                         mask=truem)

    gout, xout = gouts[li], xouts[li]
    col = t * _CAP
    aux_early = [(5, auxA.at[pl.ds(5 * _CAP, _CAP)]),
                 (6, gcxA.at[pl.ds(0, _CAP)]),
                 (7, gcyA.at[pl.ds(0, _CAP)]),
                 (8, gwA.at[pl.ds(0, _CAP)]),
                 (9, ghA.at[pl.ds(0, _CAP)]),
                 (10, labA.at[pl.ds(0, _CAP)]),
                 (11, auxA.at[pl.ds(6 * _CAP, _CAP)]),
                 (12, auxA.at[pl.ds(7 * _CAP, _CAP)])]
    for fr, src in aux_early:
      pltpu.async_copy(src, xout.at[pl.ds(fr * _ROWS + col, _CAP)], sem_w)

    def fire_body(k, c):
      for cc in range(16):
        i = k * 16 + cc
        pltpu.async_copy(yp.at[idxA.at[pl.ds(i * _CW, _CW)]],
                         gathA.at[pl.ds(i * _CW, _CW)], sem_g)
      return c
    lax.fori_loop(0, nb, fire_body, 0)

    def drain_body(k, c):
      for cc in range(16):
        pltpu.make_async_copy(yp.at[idxA.at[pl.ds(0, _CW)]],
                              gathA.at[pl.ds(0, _CW)], sem_g).wait()
      return c
    lax.fori_loop(0, nb, drain_body, 0)

    def tr_body(k, c):
      iv = lane + k * 16
      for f in range(_CW):
        pv = plsc.load_gather(gathA, [iv * _CW + f], mask=truem)
        plsc.store_scatter(gathT, [f * _CAP + iv], pv, mask=truem)
        if f < 5:
          plsc.store_scatter(auxA, [f * _CAP + iv], pv, mask=truem)
      return c
    lax.fori_loop(0, nb, tr_body, 0)

    pltpu.async_copy(gathT, gout.at[pl.ds(t * _CW * _CAP, _CW * _CAP)], sem_w)
    aux_late = [(fr, auxA.at[pl.ds(fr * _CAP, _CAP)]) for fr in range(5)]
    for fr, src in aux_late:
      pltpu.async_copy(src, xout.at[pl.ds(fr * _ROWS + col, _CAP)], sem_w)
    for fr, src in aux_early + aux_late:
      pltpu.make_async_copy(src, xout.at[pl.ds(0, _CAP)], sem_w).wait()
    pltpu.make_async_copy(gathT, gout.at[pl.ds(0, _CW * _CAP)], sem_w).wait()


def _sc_kernel_body(yp0, yp1, yp2, yt_hbm, lanes_hbm, g0, g1, g2, x0, x1, x2,
                    ytv, baseA, cellA, ordA, gcxA, gcyA, gwA, ghA, labA,
                    mapA, idxA, gathA, gathT, auxA, laneA,
                    cellS, valS, jgcx, jgcy, jgw, jgh, jlab, sem_g, sem_w):
  wid = lax.axis_index("s") * 2 + lax.axis_index("c")
  pltpu.sync_copy(lanes_hbm, laneA.at[pl.ds(0, 16)])
  lane = laneA[pl.ds(0, 16)]
  yps = (yp0, yp1, yp2)
  gouts = (g0, g1, g2)
  xouts = (x0, x1, x2)
  args = (lane, yps, yt_hbm, gouts, xouts, ytv, baseA, cellA, ordA,
          gcxA, gcyA, gwA, ghA, labA, mapA, idxA, gathA, gathT, auxA,
          cellS, valS, jgcx, jgcy, jgw, jgh, jlab, sem_g, sem_w)
  _sc_task(wid, *args)
  @pl.when(wid < _NT - 32)
  def _():
    _sc_task(wid + 32, *args)


def _sc_call(yp0f, yp1f, yp2f, yt_pad, lanes):
  mesh = plsc.VectorSubcoreMesh(core_axis_name="c", subcore_axis_name="s")
  out_type = [jax.ShapeDtypeStruct((_ROWS * _CW,), jnp.float32)
              for _ in range(3)]
  out_type += [jax.ShapeDtypeStruct((_XR * _ROWS,), jnp.float32)
               for _ in range(3)]
  f = pl.kernel(
      _sc_kernel_body,
      out_type=out_type,
      mesh=mesh,
      compiler_params=pltpu.CompilerParams(needs_layout_passes=False),
      scratch_types=[
          pltpu.VMEM((320,), jnp.float32),
          pltpu.VMEM((_FCAP,), jnp.int32),
          pltpu.VMEM((_FCAP,), jnp.int32),
          pltpu.VMEM((_FCAP,), jnp.int32),
          pltpu.VMEM((_FCAP,), jnp.float32),
          pltpu.VMEM((_FCAP,), jnp.float32),
          pltpu.VMEM((_FCAP,), jnp.float32),
          pltpu.VMEM((_FCAP,), jnp.float32),
          pltpu.VMEM((_FCAP,), jnp.float32),
          pltpu.VMEM((4096,), jnp.int32),
          pltpu.VMEM((_CAP * _CW,), jnp.int32),
          pltpu.VMEM((_CAP * _CW,), jnp.float32),
          pltpu.VMEM((_CW * _CAP,), jnp.float32),
          pltpu.VMEM((8 * _CAP,), jnp.float32),
          pltpu.VMEM((128,), jnp.int32),
          pltpu.VMEM((320,), jnp.int32),
          pltpu.VMEM((320,), jnp.int32),
          pltpu.VMEM((64,), jnp.float32),
          pltpu.VMEM((64,), jnp.float32),
          pltpu.VMEM((64,), jnp.float32),
          pltpu.VMEM((64,), jnp.float32),
          pltpu.VMEM((64,), jnp.float32),
          pltpu.SemaphoreType.DMA,
          pltpu.SemaphoreType.DMA,
      ],
  )
  return f(yp0f, yp1f, yp2f, yt_pad, lanes)


def _atan_pos(z):
  inv = z > 1.0
  t = jnp.where(inv, 1.0 / z, z)
  t2 = t * t
  p = 0.0218612288 + t2 * -0.0040540580
  p = -0.0559098861 + t2 * p
  p = 0.0964200441 + t2 * p
  p = -0.1390853351 + t2 * p
  p = 0.1994653599 + t2 * p
  p = -0.3332985605 + t2 * p
  p = 0.9999993329 + t2 * p
  p = t * p
  return jnp.where(inv, (math.pi / 2) - p, p)


def _softplus(x):
  return jnp.maximum(x, 0.0) + jnp.log1p(jnp.exp(-jnp.abs(x)))


def _tc_body(yp0, yp1, yp2, g0, g1, g2, x0, x1, x2, out):
  t = pl.program_id(0)
  a = t % 3

  @pl.when(t == 0)
  def _():
    out[...] = jnp.zeros_like(out)

  acc = jnp.zeros((8, 128), jnp.float32)
  ri = lax.broadcasted_iota(jnp.int32, (8, 128), 0)
  ci = lax.broadcasted_iota(jnp.int32, (8, 128), 1)
  grefs = (g0, g1, g2)
  xrefs = (x0, x1, x2)
  yrefs = (yp0, yp1, yp2)
  eps = 1e-9
  for li in range(3):
    s = _STRIDES[li]
    H, W = _HW[li]
    A = xrefs[li][...]
    G = grefs[li][0]
    def row(f):
      return A[f:f + 1, :]
    v = row(_FVAL) > 0.5
    vf = v.astype(jnp.float32)
    p0 = jnp.where(v, row(_FP0), 0.0)
    p1 = jnp.where(v, row(_FP1), 0.0)
    p2 = jnp.where(v, row(_FP2), 0.0)
    p3 = jnp.where(v, row(_FP3), 0.0)
    p4 = jnp.where(v, row(_FP4), 0.0)
    cell = jnp.where(v, row(_FCELL), 0.0).astype(jnp.int32)
    gcx = jnp.where(v, row(_FGCX), 0.0) / s
    gcy = jnp.where(v, row(_FGCY), 0.0) / s
    gw = jnp.where(v, row(_FGW), 1.0) / s
    gh = jnp.where(v, row(_FGH), 1.0) / s
    lab = jnp.where(v, row(_FLAB), 0.0)
    win = jnp.where(v, row(_FWIN), 0.0)
    gxf = (cell & (W - 1)).astype(jnp.float32)
    gyf = (cell >> int(math.log2(W))).astype(jnp.float32)
    aw = jnp.where(a == 0, _ANC[li][0][0],
                   jnp.where(a == 1, _ANC[li][1][0], _ANC[li][2][0]))
    ah = jnp.where(a == 0, _ANC[li][0][1],
                   jnp.where(a == 1, _ANC[li][1][1], _ANC[li][2][1]))
    px = 1.0 / (1.0 + jnp.exp(-p0)) + gxf
    py = 1.0 / (1.0 + jnp.exp(-p1)) + gyf
    pw = jnp.exp(p2) * aw
    ph = jnp.exp(p3) * ah
    b1x1 = px - pw * 0.5; b1x2 = px + pw * 0.5
    b1y1 = py - ph * 0.5; b1y2 = py + ph * 0.5
    b2x1 = gcx - gw * 0.5; b2x2 = gcx + gw * 0.5
    b2y1 = gcy - gh * 0.5; b2y2 = gcy + gh * 0.5
    inter = (jnp.maximum(jnp.minimum(b1x2, b2x2) - jnp.maximum(b1x1, b2x1), 0.0)
             * jnp.maximum(jnp.minimum(b1y2, b2y2) - jnp.maximum(b1y1, b2y1),
                           0.0))
    w1 = b1x2 - b1x1; h1 = b1y2 - b1y1 + eps
    w2 = b2x2 - b2x1; h2 = b2y2 - b2y1 + eps
    union = w1 * h1 + w2 * h2 - inter + eps
    iou = inter / union
    cw_ = jnp.maximum(b1x2, b2x2) - jnp.minimum(b1x1, b2x1)
    ch_ = jnp.maximum(b1y2, b2y2) - jnp.minimum(b1y1, b2y1)
    c2 = cw_ * cw_ + ch_ * ch_ + eps
    rho2 = ((b2x1 + b2x2 - b1x1 - b1x2) ** 2
            + (b2y1 + b2y2 - b1y1 - b1y2) ** 2) / 4.0
    vv = (4.0 / (math.pi ** 2)) * (_atan_pos(w2 / h2) - _atan_pos(w1 / h1)) ** 2
    alpha = vv / (1.0 + eps - iou + vv)
    ciou = iou - (rho2 / c2 + vv * alpha)

    box_sum = jnp.sum(vf * (1.0 - ciou))
    cnt = jnp.sum(vf)
    val = jnp.maximum(ciou, 0.0)
    corr = jnp.sum(vf * win * val * p4)

    chan = lax.broadcasted_iota(jnp.int32, (_CW, _CAP), 0)
    cmask = (chan >= 5) & (chan < 85) & v
    gx_ = jnp.where(cmask, G, 0.0)
    sp = jnp.where(cmask, _softplus(gx_), 0.0)
    pick = jnp.where(cmask & ((chan - 5) == lab.astype(jnp.int32)), gx_, 0.0)
    cls_sum = jnp.sum(sp) - jnp.sum(pick)

    plane = yrefs[li][...]
    spo = jnp.sum(_softplus(plane))

    for k, sv in enumerate((box_sum, cnt, cls_sum, corr, spo)):
      acc = acc + jnp.where((ri == li) & (ci == k), sv, 0.0)
  out[...] += acc


def _tc_call(yp0, yp1, yp2, gs, xs):
  grid = (_NT,)
  in_specs = [
      pl.BlockSpec((64 * 64,), lambda t: ((t // 3) * 255 + 85 * (t % 3) + 4,)),
      pl.BlockSpec((32 * 32,), lambda t: ((t // 3) * 255 + 85 * (t % 3) + 4,)),
      pl.BlockSpec((16 * 16,), lambda t: ((t // 3) * 255 + 85 * (t % 3) + 4,)),
      pl.BlockSpec((1, _CW, _CAP), lambda t: (t, 0, 0)),
      pl.BlockSpec((1, _CW, _CAP), lambda t: (t, 0, 0)),
      pl.BlockSpec((1, _CW, _CAP), lambda t: (t, 0, 0)),
      pl.BlockSpec((_XR, _CAP), lambda t: (0, t)),
      pl.BlockSpec((_XR, _CAP), lambda t: (0, t)),
      pl.BlockSpec((_XR, _CAP), lambda t: (0, t)),
  ]
  out_spec = pl.BlockSpec((8, 128), lambda t: (0, 0))
  return pl.pallas_call(
      _tc_body,
      grid=grid,
      in_specs=in_specs,
      out_specs=out_spec,
      out_shape=jax.ShapeDtypeStruct((8, 128), jnp.float32),
      compiler_params=pltpu.CompilerParams(
          dimension_semantics=("arbitrary",)),
  )(yp0, yp1, yp2, gs[0].reshape(_NT, _CW, _CAP), gs[1].reshape(_NT, _CW, _CAP),
    gs[2].reshape(_NT, _CW, _CAP), xs[0].reshape(_XR, _ROWS),
    xs[1].reshape(_XR, _ROWS), xs[2].reshape(_XR, _ROWS))


def kernel(y_pred_0, y_pred_1, y_pred_2, y_true):
  yt = y_true.reshape(_B, _L * 5)
  yt_pad = jnp.pad(yt, ((0, 0), (0, 320 - _L * 5)))
  lanes = jnp.arange(16, dtype=jnp.int32)
  yp0f = y_pred_0.reshape(-1)
  yp1f = y_pred_1.reshape(-1)
  yp2f = y_pred_2.reshape(-1)
  outs = _sc_call(yp0f, yp1f, yp2f, yt_pad, lanes)
  gs, xs = outs[:3], outs[3:]
  acc = _tc_call(yp0f, yp1f, yp2f, gs, xs)

  loss_box = jnp.float32(0.0)
  loss_obj = jnp.float32(0.0)
  loss_cls = jnp.float32(0.0)
  for li in range(3):
    H, W = _HW[li]
    box_sum = acc[li, 0]
    cnt = acc[li, 1]
    cls_sum = acc[li, 2]
    corr = acc[li, 3]
    spo = acc[li, 4]
    cntf = jnp.maximum(cnt, 1.0)
    has = cnt > 0
    loss_box = loss_box + jnp.where(has, box_sum / cntf, 0.0)
    loss_cls = loss_cls + jnp.where(has, cls_sum / (cntf * _NCLS), 0.0)
    loss_obj = loss_obj + (spo - corr) / (_B * 3 * H * W)
  total = _WBOX * loss_box + _WOBJ * loss_obj + _WCLS * loss_cls
  return total, _WBOX * loss_box, _WOBJ * loss_obj, _WCLS * loss_cls
